# Initial kernel scaffold; baseline (speedup 1.0000x reference)
import jax
import jax.numpy as jnp
from jax.experimental import pallas as pl


def kernel(x, edge_index, batch, W1, b1, W2, b2, W3, b3, g1, be1, g2, be2, g3, be3, fcW1, fcb1, fcW2, fcb2):
    raise NotImplementedError("write your pallas kernel here")


# speedup vs baseline: 16.3922x; 16.3922x over previous
import functools

import jax
import jax.numpy as jnp
from jax.experimental import pallas as pl
from jax.experimental.pallas import tpu as pltpu
from jax.experimental.pallas import tpu_sc as plsc

N = 10000
E = 320000
IN_CH = 128
HID = 64
G = 64
EPS = 1e-5

NC = 2
NS = 16
NW = NC * NS
CHUNK = 384
CH0 = 36
CH1 = 17
CHMAX = max(CH0, CH1)
TOT_CHKS = NS * (CH0 + CH1)
NPAD = 10240
RPT = NPAD // NS
DEGW = 16


@functools.lru_cache(maxsize=None)
def _sc_kernels():
    mesh = plsc.VectorSubcoreMesh(
        core_axis_name="c", subcore_axis_name="s", num_cores=NC, num_subcores=NS
    )
    params = pltpu.CompilerParams(use_tc_tiling_on_sc=False)

    @functools.partial(
        pl.kernel,
        out_type=jax.ShapeDtypeStruct((NC * NPAD, DEGW), jnp.float32),
        mesh=mesh,
        scratch_types=[
            pltpu.VMEM((CHMAX, CHUNK), jnp.int32),
            pltpu.VMEM((CHUNK, DEGW), jnp.float32),
            pltpu.VMEM_SHARED((NPAD, DEGW), jnp.float32),
            pltpu.SemaphoreType.DMA,
        ],
        compiler_params=params,
    )
    def sc_degree(dst_hbm, zeros_hbm, ones_hbm, out_hbm, dst_v, ones_v, acc,
                  sem):
        c = jax.lax.axis_index("c")
        s = jax.lax.axis_index("s")
        wid = c * NS + s
        nch = jnp.where(c == 0, CH0, CH1)
        pltpu.sync_copy(dst_hbm.at[wid], dst_v)
        pltpu.sync_copy(ones_hbm, ones_v)
        pltpu.sync_copy(
            zeros_hbm.at[pl.ds(s * RPT, RPT)], acc.at[pl.ds(s * RPT, RPT)]
        )
        plsc.subcore_barrier()

        def body(j, carry):
            pltpu.sync_copy(ones_v, acc.at[dst_v.at[j]], add=True)
            return carry

        jax.lax.fori_loop(0, nch, body, 0)
        plsc.subcore_barrier()
        pltpu.sync_copy(
            acc.at[pl.ds(s * RPT, RPT)],
            out_hbm.at[pl.ds(c * NPAD + s * RPT, RPT)],
        )

    @functools.partial(
        pl.kernel,
        out_type=jax.ShapeDtypeStruct((NC * NPAD, HID), jnp.float32),
        mesh=mesh,
        scratch_types=[
            pltpu.VMEM((CHMAX, CHUNK), jnp.int32),
            pltpu.VMEM((CHMAX, CHUNK), jnp.int32),
            pltpu.VMEM((CHUNK, HID), jnp.float32),
            pltpu.VMEM_SHARED((NPAD, HID), jnp.float32),
            pltpu.SemaphoreType.DMA,
        ],
        compiler_params=params,
    )
    def sc_aggregate(xs_hbm, src_hbm, dst_hbm, zeros_hbm, out_hbm,
                     src_v, dst_v, rows_v, acc, sem):
        c = jax.lax.axis_index("c")
        s = jax.lax.axis_index("s")
        wid = c * NS + s
        nch = jnp.where(c == 0, CH0, CH1)
        pltpu.sync_copy(src_hbm.at[wid], src_v)
        pltpu.sync_copy(dst_hbm.at[wid], dst_v)
        pltpu.sync_copy(
            zeros_hbm.at[pl.ds(s * RPT, RPT)], acc.at[pl.ds(s * RPT, RPT)]
        )
        plsc.subcore_barrier()

        def body(j, carry):
            pltpu.async_copy(xs_hbm.at[src_v.at[j]], rows_v, sem).wait()
            pltpu.sync_copy(rows_v, acc.at[dst_v.at[j]], add=True)
            return carry

        jax.lax.fori_loop(0, nch, body, 0)
        plsc.subcore_barrier()
        pltpu.sync_copy(
            acc.at[pl.ds(s * RPT, RPT)],
            out_hbm.at[pl.ds(c * NPAD + s * RPT, RPT)],
        )

    return sc_degree, sc_aggregate


def _sc_degree(dst_b, zeros16, ones16):
    return _sc_kernels()[0](dst_b, zeros16, ones16)


def _sc_aggregate(xs, src_b, dst_b, zeros64):
    return _sc_kernels()[1](xs, src_b, dst_b, zeros64)


def _dis_from_degp(degp):
    deg = degp[0, :N, 0:1] + degp[1, :N, 0:1] + 1.0
    return jax.lax.rsqrt(deg)


def _tc_prep_body(x_ref, w_ref, degp_ref, out_ref):
    dis = _dis_from_degp(degp_ref[...])
    xl = jnp.dot(x_ref[...], w_ref[...], preferred_element_type=jnp.float32)
    out_ref[0:N, :] = xl * dis
    out_ref[N:NPAD, :] = jnp.zeros((NPAD - N, HID), jnp.float32)


def _bn_relu(aggp, xs, degp, b, g, be):
    dis = _dis_from_degp(degp)
    t = (aggp[0, :N, :] + aggp[1, :N, :] + xs[:N, :]) * dis + b
    mu = jnp.mean(t, axis=0, keepdims=True)
    var = jnp.mean((t - mu) ** 2, axis=0, keepdims=True)
    h = (t - mu) * jax.lax.rsqrt(var + EPS) * g + be
    return jnp.maximum(h, 0.0)


def _tc_mid_body(aggp_ref, xs_ref, degp_ref, b_ref, g_ref, be_ref, w_ref,
                 out_ref):
    h = _bn_relu(aggp_ref[...], xs_ref[...], degp_ref[...],
                 b_ref[...], g_ref[...], be_ref[...])
    dis = _dis_from_degp(degp_ref[...])
    xl = jnp.dot(h, w_ref[...], preferred_element_type=jnp.float32)
    out_ref[0:N, :] = xl * dis
    out_ref[N:NPAD, :] = jnp.zeros((NPAD - N, HID), jnp.float32)


def _tc_head_body(aggp_ref, xs_ref, degp_ref, b_ref, g_ref, be_ref,
                  batch_ref, fw1_ref, fb1_ref, fw2_ref, fb2_ref, out_ref):
    h = _bn_relu(aggp_ref[...], xs_ref[...], degp_ref[...],
                 b_ref[...], g_ref[...], be_ref[...])
    gid = jax.lax.broadcasted_iota(jnp.int32, (N, G), 1)
    onehot = (batch_ref[...] == gid).astype(jnp.float32)
    dn = (((0,), (0,)), ((), ()))
    sums = jax.lax.dot_general(onehot, h, dn,
                               preferred_element_type=jnp.float32)
    cnts = jax.lax.dot_general(onehot, jnp.ones((N, 1), jnp.float32), dn,
                               preferred_element_type=jnp.float32)
    pooled = sums / jnp.maximum(cnts, 1.0)
    h2 = jnp.maximum(
        jnp.dot(pooled, fw1_ref[...], preferred_element_type=jnp.float32)
        + fb1_ref[...], 0.0)
    o = jnp.dot(h2, fw2_ref[...], preferred_element_type=jnp.float32) \
        + fb2_ref[...]
    out_ref[...] = 1.0 / (1.0 + jnp.exp(-o))


_tc_prep = pl.pallas_call(
    _tc_prep_body,
    out_shape=jax.ShapeDtypeStruct((NPAD, HID), jnp.float32),
)

_tc_mid = pl.pallas_call(
    _tc_mid_body,
    out_shape=jax.ShapeDtypeStruct((NPAD, HID), jnp.float32),
)

_tc_head = pl.pallas_call(
    _tc_head_body,
    out_shape=jax.ShapeDtypeStruct((G, 2), jnp.float32),
)


def kernel(x, edge_index, batch, W1, b1, W2, b2, W3, b3, g1, be1, g2, be2,
           g3, be3, fcW1, fcb1, fcW2, fcb2):
    src = edge_index[0].astype(jnp.int32)
    dst = edge_index[1].astype(jnp.int32)
    def _split(idx):
        flat = jnp.concatenate(
            [idx, jnp.full((TOT_CHKS * CHUNK - E,), N, jnp.int32)])
        c0 = flat[: NS * CH0 * CHUNK].reshape(NS, CH0, CHUNK)
        c1 = flat[NS * CH0 * CHUNK:].reshape(NS, CH1, CHUNK)
        c0 = jnp.concatenate(
            [c0, jnp.full((NS, CHMAX - CH0, CHUNK), N, jnp.int32)], axis=1)
        c1 = jnp.concatenate(
            [c1, jnp.full((NS, CHMAX - CH1, CHUNK), N, jnp.int32)], axis=1)
        return jnp.concatenate([c0, c1], axis=0)

    src_b = _split(src)
    dst_b = _split(dst)

    zeros64 = jnp.zeros((NPAD, HID), jnp.float32)
    zeros16 = jnp.zeros((NPAD, DEGW), jnp.float32)
    ones16 = jnp.ones((CHUNK, DEGW), jnp.float32)

    degp = _sc_degree(dst_b, zeros16, ones16).reshape(NC, NPAD, DEGW)

    b1r, b2r, b3r = b1.reshape(1, HID), b2.reshape(1, HID), b3.reshape(1, HID)
    g1r, g2r, g3r = g1.reshape(1, HID), g2.reshape(1, HID), g3.reshape(1, HID)
    be1r, be2r, be3r = (be1.reshape(1, HID), be2.reshape(1, HID),
                        be3.reshape(1, HID))

    xs1 = _tc_prep(x, W1, degp)
    agg1 = _sc_aggregate(xs1, src_b, dst_b, zeros64).reshape(NC, NPAD, HID)
    xs2 = _tc_mid(agg1, xs1, degp, b1r, g1r, be1r, W2)
    agg2 = _sc_aggregate(xs2, src_b, dst_b, zeros64).reshape(NC, NPAD, HID)
    xs3 = _tc_mid(agg2, xs2, degp, b2r, g2r, be2r, W3)
    agg3 = _sc_aggregate(xs3, src_b, dst_b, zeros64).reshape(NC, NPAD, HID)

    batch2d = batch.astype(jnp.int32).reshape(N, 1)
    return _tc_head(agg3, xs3, degp, b3r, g3r, be3r, batch2d,
                    fcW1, fcb1.reshape(1, 32), fcW2, fcb2.reshape(1, 2))
